# Initial kernel scaffold; baseline (speedup 1.0000x reference)
import jax
import jax.numpy as jnp
from jax.experimental import pallas as pl


def kernel(x, weight):
    raise NotImplementedError("write your pallas kernel here")


# speedup vs baseline: 1.5767x; 1.5767x over previous
import functools
import jax
import jax.numpy as jnp
from jax import lax
from jax.experimental import pallas as pl
from jax.experimental.pallas import tpu as pltpu
from jax.experimental.pallas import tpu_sc as plsc


def _make_gather(B, D, CH):
    info = plsc.get_sparse_core_info()
    NC, NS = info.num_cores, info.num_subcores
    NW = NC * NS
    b_per_w = B // NW
    n_ch = b_per_w // CH
    assert n_ch * CH == b_per_w
    mesh = plsc.VectorSubcoreMesh(core_axis_name="c", subcore_axis_name="s")

    @functools.partial(
        pl.kernel,
        mesh=mesh,
        out_type=jax.ShapeDtypeStruct((B, D), jnp.float32),
        compiler_params=pltpu.CompilerParams(use_tc_tiling_on_sc=False),
        scratch_types=[
            pltpu.VMEM((CH,), jnp.int32),
            pltpu.VMEM((CH,), jnp.int32),
            pltpu.VMEM((CH, D), jnp.float32),
            pltpu.VMEM((CH, D), jnp.float32),
            pltpu.SemaphoreType.DMA,
            pltpu.SemaphoreType.DMA,
            pltpu.SemaphoreType.DMA,
            pltpu.SemaphoreType.DMA,
        ],
    )
    def k(idx_hbm, w_hbm, out_hbm, idx0, idx1, rows0, rows1,
          g0, g1, o0, o1):
        wid = lax.axis_index("s") * NC + lax.axis_index("c")
        base = wid * b_per_w
        idx_bufs = (idx0, idx1)
        row_bufs = (rows0, rows1)
        gsems = (g0, g1)
        osems = (o0, o1)

        pltpu.sync_copy(idx_hbm.at[pl.ds(base, CH)], idx0)
        gathers = [pltpu.async_copy(w_hbm.at[idx0], rows0, g0)]
        outs = [None] * n_ch
        for i in range(n_ch):
            cur = i % 2
            nxt = (i + 1) % 2
            if i + 1 < n_ch:
                pltpu.sync_copy(
                    idx_hbm.at[pl.ds(base + (i + 1) * CH, CH)], idx_bufs[nxt])
                if i >= 1:
                    outs[i - 1].wait()
                gathers.append(
                    pltpu.async_copy(w_hbm.at[idx_bufs[nxt]],
                                     row_bufs[nxt], gsems[nxt]))
            gathers[i].wait()
            outs[i] = pltpu.async_copy(
                row_bufs[cur], out_hbm.at[pl.ds(base + i * CH, CH)],
                osems[cur])
        if n_ch >= 2:
            outs[n_ch - 2].wait()
        outs[n_ch - 1].wait()

    return k


def kernel(x, weight):
    B = x.shape[0] * x.shape[1]
    D = weight.shape[1]
    flat_idx = x.reshape(B).astype(jnp.int32)
    gather = _make_gather(B, D, 1024)
    out = gather(flat_idx, weight)
    return out.reshape(x.shape[0], x.shape[1], D)


# speedup vs baseline: 1.6594x; 1.0524x over previous
import functools
import jax
import jax.numpy as jnp
from jax import lax
from jax.experimental import pallas as pl
from jax.experimental.pallas import tpu as pltpu
from jax.experimental.pallas import tpu_sc as plsc


def _make_gather(C, B, D):
    info = plsc.get_sparse_core_info()
    NC, NS = info.num_cores, info.num_subcores
    NW = NC * NS
    CH = B // NW
    K = 2
    n_grp = C // K
    assert n_grp * K == C
    mesh = plsc.VectorSubcoreMesh(core_axis_name="c", subcore_axis_name="s")

    @functools.partial(
        pl.kernel,
        mesh=mesh,
        out_type=jax.ShapeDtypeStruct((C, B, D), jnp.float32),
        compiler_params=pltpu.CompilerParams(use_tc_tiling_on_sc=False),
        scratch_types=[
            pltpu.VMEM((CH,), jnp.int32),
            pltpu.VMEM((CH,), jnp.int32),
            pltpu.VMEM((CH, D), jnp.float32),
            pltpu.VMEM((CH, D), jnp.float32),
            pltpu.SemaphoreType.DMA,
            pltpu.SemaphoreType.DMA,
            pltpu.SemaphoreType.DMA,
            pltpu.SemaphoreType.DMA,
        ],
    )
    def k(xt_hbm, w_hbm, out_hbm, idx0, idx1, rows0, rows1, g0, g1, o0, o1):
        wid = lax.axis_index("s") * NC + lax.axis_index("c")
        base = wid * CH
        idx_bufs = (idx0, idx1)
        row_bufs = (rows0, rows1)
        gsems = (g0, g1)
        osems = (o0, o1)

        @pl.loop(0, n_grp)
        def grp(i):
            c0 = i * K
            gathers = []
            for b in range(K):
                @pl.when(i > 0)
                def _():
                    pltpu.make_async_copy(
                        row_bufs[b],
                        out_hbm.at[c0 + b - K, pl.ds(base, CH)],
                        osems[b]).wait()
                pltpu.sync_copy(xt_hbm.at[c0 + b, pl.ds(base, CH)],
                                idx_bufs[b])
                gathers.append(
                    pltpu.async_copy(w_hbm.at[idx_bufs[b]], row_bufs[b],
                                     gsems[b]))
            for b in range(K):
                gathers[b].wait()
                pltpu.async_copy(row_bufs[b],
                                 out_hbm.at[c0 + b, pl.ds(base, CH)],
                                 osems[b])

        for b in range(K):
            pltpu.make_async_copy(
                row_bufs[b],
                out_hbm.at[C - K + b, pl.ds(base, CH)],
                osems[b]).wait()

    return k


def kernel(x, weight):
    B, C = x.shape
    D = weight.shape[1]
    xt = x.T.astype(jnp.int32)
    gather = _make_gather(C, B, D)
    out = gather(xt, weight)
    return jnp.transpose(out, (1, 0, 2))
